# baseline (speedup 1.0000x reference)
import jax
import jax.numpy as jnp
from jax import lax
from jax.experimental import pallas as pl
from jax.experimental.pallas import tpu as pltpu
from jax.experimental.pallas import tpu_sc as plsc

N_NODES = 10000
N_PAD = 10240
N_EDGES = 320000
H = 128

NC = 2
NS = 16
NW = NC * NS
EDGES_PER_TILE = N_EDGES // NW
CHUNK = 40
N_CHUNKS = EDGES_PER_TILE // CHUNK
ROWS_PER_TILE = N_PAD // NS
NBUF = 5


def _mesh():
    return plsc.VectorSubcoreMesh(
        core_axis_name="c", subcore_axis_name="s", num_cores=NC, num_subcores=NS
    )


def _neg_relu_mm_body(e_ref, w_ref, o_ref):
    e = jnp.maximum(e_ref[...], 0.0)
    o_ref[...] = -lax.dot_general(
        e, w_ref[...], (((1,), (1,)), ((), ())),
        preferred_element_type=jnp.float32,
        precision=lax.Precision.DEFAULT,
    )


def _neg_relu_mm(E, W):
    blk = 1600
    return pl.pallas_call(
        _neg_relu_mm_body,
        grid=(N_EDGES // blk,),
        in_specs=[
            pl.BlockSpec((blk, H), lambda i: (i, 0)),
            pl.BlockSpec((H, H), lambda i: (0, 0)),
        ],
        out_specs=pl.BlockSpec((blk, H), lambda i: (i, 0)),
        out_shape=jax.ShapeDtypeStruct((N_EDGES, H), jnp.float32),
    )(E, W)


def _combine_body(p_ref, w_ref, b_ref, o_ref):
    m = p_ref[0] + p_ref[1]
    o_ref[...] = b_ref[...] + lax.dot_general(
        m, w_ref[...], (((1,), (1,)), ((), ())),
        preferred_element_type=jnp.float32,
        precision=lax.Precision.DEFAULT,
    )


def _combine(parts, W, b):
    blk = 2048
    return pl.pallas_call(
        _combine_body,
        grid=(N_PAD // blk,),
        in_specs=[
            pl.BlockSpec((NC, blk, H), lambda i: (0, i, 0)),
            pl.BlockSpec((H, H), lambda i: (0, 0)),
            pl.BlockSpec((1, H), lambda i: (0, 0)),
        ],
        out_specs=pl.BlockSpec((blk, H), lambda i: (i, 0)),
        out_shape=jax.ShapeDtypeStruct((N_PAD, H), jnp.float32),
    )(parts, W, b.reshape(1, H))


def _scatter_body(e_hbm, dest_hbm, zeros_hbm, out_hbm, didx, bufs, acc, *sems):
    semI = sems[0:NBUF]
    semR = sems[NBUF:2 * NBUF]
    semS = sems[2 * NBUF:3 * NBUF]
    c = lax.axis_index("c")
    s = lax.axis_index("s")
    wid = c * NS + s
    row0 = s * ROWS_PER_TILE
    base = wid * EDGES_PER_TILE

    def idx_src(j):
        return dest_hbm.at[pl.ds(base + j * CHUNK, CHUNK)]

    def rows_src(j):
        return e_hbm.at[pl.ds(base + j * CHUNK, CHUNK)]

    def start_I(j, b):
        pltpu.async_copy(idx_src(j), didx.at[b], semI[b])

    def wait_I(j, b):
        pltpu.make_async_copy(idx_src(j), didx.at[b], semI[b]).wait()

    def start_R(j, b):
        pltpu.async_copy(rows_src(j), bufs.at[b], semR[b])

    def wait_R(j, b):
        pltpu.make_async_copy(rows_src(j), bufs.at[b], semR[b]).wait()

    def start_S(j, b):
        pltpu.async_copy(bufs.at[b], acc.at[didx.at[b]], semS[b], add=True)

    def wait_S(j, b):
        pltpu.make_async_copy(bufs.at[b], acc.at[didx.at[b]], semS[b]).wait()

    def relu_buf(b):
        def rb(r, _):
            for rr in range(4):
                for cc in range(H // 16):
                    sl = (b, r * 4 + rr, pl.ds(cc * 16, 16))
                    bufs[sl] = jnp.maximum(bufs[sl], 0.0)
            return 0
        lax.fori_loop(0, CHUNK // 4, rb, 0)

    for k in range(3):
        start_I(k, k)
    pltpu.sync_copy(
        zeros_hbm.at[pl.ds(row0, ROWS_PER_TILE)], acc.at[pl.ds(row0, ROWS_PER_TILE)]
    )
    plsc.subcore_barrier()
    for k in range(2):
        wait_I(k, k)
        start_R(k, k)

    def group(g, _):
        for b0 in range(NBUF):
            j = g * NBUF + b0

            @pl.when(j >= 2)
            def _():
                wait_S(j - 2, (b0 - 2) % NBUF)

            @pl.when(j + 2 < N_CHUNKS)
            def _():
                wait_I(j + 2, (b0 + 2) % NBUF)

            @pl.when(j + 3 < N_CHUNKS)
            def _():
                start_I(j + 3, (b0 + 3) % NBUF)

            @pl.when(j + 2 < N_CHUNKS)
            def _():
                start_R(j + 2, (b0 + 2) % NBUF)

            wait_R(j, b0)
            relu_buf(b0)
            start_S(j, b0)
        return 0

    lax.fori_loop(0, N_CHUNKS // NBUF, group, 0)
    for j in (N_CHUNKS - 2, N_CHUNKS - 1):
        wait_S(j, j % NBUF)

    plsc.subcore_barrier()
    pltpu.sync_copy(
        acc.at[pl.ds(row0, ROWS_PER_TILE)],
        out_hbm.at[c, pl.ds(row0, ROWS_PER_TILE)],
    )


def _scatter_partials(E, dest, zeros):
    return pl.kernel(
        _scatter_body,
        out_type=jax.ShapeDtypeStruct((NC, N_PAD, H), jnp.float32),
        mesh=_mesh(),
        scratch_types=[
            pltpu.VMEM((NBUF, CHUNK), jnp.int32),
            pltpu.VMEM((NBUF, CHUNK, H), jnp.float32),
            pltpu.VMEM_SHARED((N_PAD, H), jnp.float32),
        ]
        + [pltpu.SemaphoreType.DMA] * (3 * NBUF),
    )(E, dest, zeros)


def _gather_body(a_hbm, np_hbm, src_hbm, rev_hbm, out_hbm, sidx, ridx, bufs, tbl,
                 *sems):
    semIs = sems[0:NBUF]
    semIr = sems[NBUF:2 * NBUF]
    semA = sems[2 * NBUF:3 * NBUF]
    semW = sems[3 * NBUF:4 * NBUF]
    semD = sems[4 * NBUF:]
    c = lax.axis_index("c")
    s = lax.axis_index("s")
    wid = c * NS + s
    row0 = s * ROWS_PER_TILE
    base = wid * EDGES_PER_TILE

    def out_dst(j):
        return out_hbm.at[pl.ds(base + j * CHUNK, CHUNK)]

    def start_Is(j, b):
        pltpu.async_copy(src_hbm.at[pl.ds(base + j * CHUNK, CHUNK)], sidx.at[b],
                         semIs[b])

    def wait_Is(j, b):
        pltpu.make_async_copy(src_hbm.at[pl.ds(base + j * CHUNK, CHUNK)],
                              sidx.at[b], semIs[b]).wait()

    def start_Ir(j, b):
        pltpu.async_copy(rev_hbm.at[pl.ds(base + j * CHUNK, CHUNK)], ridx.at[b],
                         semIr[b])

    def wait_Ir(j, b):
        pltpu.make_async_copy(rev_hbm.at[pl.ds(base + j * CHUNK, CHUNK)],
                              ridx.at[b], semIr[b]).wait()

    def start_A(j, b):
        pltpu.async_copy(tbl.at[sidx.at[b]], bufs.at[b], semA[b])

    def wait_A(j, b):
        pltpu.make_async_copy(tbl.at[sidx.at[b]], bufs.at[b], semA[b]).wait()

    def start_D(j, b):
        pltpu.async_copy(np_hbm.at[ridx.at[b]], bufs.at[b], semD[b], add=True)

    def wait_D(j, b):
        pltpu.make_async_copy(np_hbm.at[ridx.at[b]], bufs.at[b], semD[b]).wait()

    def start_W(j, b):
        pltpu.async_copy(bufs.at[b], out_dst(j), semW[b])

    def wait_W(j, b):
        pltpu.make_async_copy(bufs.at[b], out_dst(j), semW[b]).wait()

    for k in range(2):
        start_Is(k, k)
        start_Ir(k, k)
    pltpu.sync_copy(
        a_hbm.at[pl.ds(row0, ROWS_PER_TILE)], tbl.at[pl.ds(row0, ROWS_PER_TILE)]
    )
    plsc.subcore_barrier()
    wait_Is(0, 0)
    wait_Ir(0, 0)
    start_A(0, 0)

    def group(g, _):
        for b0 in range(NBUF):
            j = g * NBUF + b0

            @pl.when(j >= 4)
            def _():
                wait_W(j - 4, (b0 - 4) % NBUF)

            @pl.when(j >= 3)
            def _():
                wait_D(j - 3, (b0 - 3) % NBUF)
                start_W(j - 3, (b0 - 3) % NBUF)

            @pl.when(j + 1 < N_CHUNKS)
            def _():
                wait_Is(j + 1, (b0 + 1) % NBUF)
                wait_Ir(j + 1, (b0 + 1) % NBUF)

            @pl.when(j + 2 < N_CHUNKS)
            def _():
                start_Is(j + 2, (b0 + 2) % NBUF)
                start_Ir(j + 2, (b0 + 2) % NBUF)

            @pl.when(j + 1 < N_CHUNKS)
            def _():
                start_A(j + 1, (b0 + 1) % NBUF)

            wait_A(j, b0)
            start_D(j, b0)
        return 0

    lax.fori_loop(0, N_CHUNKS // NBUF, group, 0)
    for j in (N_CHUNKS - 3, N_CHUNKS - 2, N_CHUNKS - 1):
        wait_D(j, j % NBUF)
        start_W(j, j % NBUF)
    for j in range(N_CHUNKS - 4, N_CHUNKS):
        wait_W(j, j % NBUF)


def _gather_combine(A, nP, src, rev):
    return pl.kernel(
        _gather_body,
        out_type=jax.ShapeDtypeStruct((N_EDGES, H), jnp.float32),
        mesh=_mesh(),
        scratch_types=[
            pltpu.VMEM((NBUF, CHUNK), jnp.int32),
            pltpu.VMEM((NBUF, CHUNK), jnp.int32),
            pltpu.VMEM((NBUF, CHUNK, H), jnp.float32),
            pltpu.VMEM_SHARED((N_PAD, H), jnp.float32),
        ]
        + [pltpu.SemaphoreType.DMA] * (5 * NBUF),
    )(A, nP, src, rev)


def kernel(V, E, edge_index, rev_index, W, b):
    src = edge_index[0]
    dest = edge_index[1]
    zeros = jnp.zeros((N_PAD, H), jnp.float32)
    parts = _scatter_partials(E, dest, zeros)
    nP = _neg_relu_mm(E, W)
    A = _combine(parts, W, b)
    return _gather_combine(A, nP, src, rev_index)


# speedup vs baseline: 1.0496x; 1.0496x over previous
import jax
import jax.numpy as jnp
from jax import lax
from jax.experimental import pallas as pl
from jax.experimental.pallas import tpu as pltpu
from jax.experimental.pallas import tpu_sc as plsc

N_NODES = 10000
N_PAD = 10240
N_EDGES = 320000
H = 128

NC = 2
NS = 16
NW = NC * NS
EDGES_PER_TILE = N_EDGES // NW
CHUNK = 40
N_CHUNKS = EDGES_PER_TILE // CHUNK
ROWS_PER_TILE = N_PAD // NS
NBUF = 5


def _mesh():
    return plsc.VectorSubcoreMesh(
        core_axis_name="c", subcore_axis_name="s", num_cores=NC, num_subcores=NS
    )


def _neg_relu_mm_body(e_ref, w_ref, o_ref):
    e = jnp.maximum(e_ref[...], 0.0)
    o_ref[...] = -lax.dot_general(
        e, w_ref[...], (((1,), (1,)), ((), ())),
        preferred_element_type=jnp.float32,
        precision=lax.Precision.DEFAULT,
    )


def _neg_relu_mm(E, W):
    blk = 3200
    return pl.pallas_call(
        _neg_relu_mm_body,
        grid=(N_EDGES // blk,),
        in_specs=[
            pl.BlockSpec((blk, H), lambda i: (i, 0)),
            pl.BlockSpec((H, H), lambda i: (0, 0)),
        ],
        out_specs=pl.BlockSpec((blk, H), lambda i: (i, 0)),
        out_shape=jax.ShapeDtypeStruct((N_EDGES, H), jnp.float32),
    )(E, W)


def _combine_body(p_ref, w_ref, b_ref, o_ref):
    m = p_ref[0] + p_ref[1]
    o_ref[...] = b_ref[...] + lax.dot_general(
        m, w_ref[...], (((1,), (1,)), ((), ())),
        preferred_element_type=jnp.float32,
        precision=lax.Precision.DEFAULT,
    )


def _combine(parts, W, b):
    blk = 2048
    return pl.pallas_call(
        _combine_body,
        grid=(N_PAD // blk,),
        in_specs=[
            pl.BlockSpec((NC, blk, H), lambda i: (0, i, 0)),
            pl.BlockSpec((H, H), lambda i: (0, 0)),
            pl.BlockSpec((1, H), lambda i: (0, 0)),
        ],
        out_specs=pl.BlockSpec((blk, H), lambda i: (i, 0)),
        out_shape=jax.ShapeDtypeStruct((N_PAD, H), jnp.float32),
    )(parts, W, b.reshape(1, H))


def _scatter_body(e_hbm, dest_hbm, zeros_hbm, out_hbm, didx, bufs, acc, *sems):
    semI = sems[0:NBUF]
    semR = sems[NBUF:2 * NBUF]
    semS = sems[2 * NBUF:3 * NBUF]
    c = lax.axis_index("c")
    s = lax.axis_index("s")
    wid = c * NS + s
    row0 = s * ROWS_PER_TILE
    base = wid * EDGES_PER_TILE

    def idx_src(j):
        return dest_hbm.at[pl.ds(base + j * CHUNK, CHUNK)]

    def rows_src(j):
        return e_hbm.at[pl.ds(base + j * CHUNK, CHUNK)]

    def start_I(j, b):
        pltpu.async_copy(idx_src(j), didx.at[b], semI[b])

    def wait_I(j, b):
        pltpu.make_async_copy(idx_src(j), didx.at[b], semI[b]).wait()

    def start_R(j, b):
        pltpu.async_copy(rows_src(j), bufs.at[b], semR[b])

    def wait_R(j, b):
        pltpu.make_async_copy(rows_src(j), bufs.at[b], semR[b]).wait()

    def start_S(j, b):
        pltpu.async_copy(bufs.at[b], acc.at[didx.at[b]], semS[b], add=True)

    def wait_S(j, b):
        pltpu.make_async_copy(bufs.at[b], acc.at[didx.at[b]], semS[b]).wait()

    def relu_buf(b):
        def rb(r, _):
            for rr in range(4):
                for cc in range(H // 16):
                    sl = (b, r * 4 + rr, pl.ds(cc * 16, 16))
                    bufs[sl] = jnp.maximum(bufs[sl], 0.0)
            return 0
        lax.fori_loop(0, CHUNK // 4, rb, 0)

    for k in range(3):
        start_I(k, k)
    pltpu.sync_copy(
        zeros_hbm.at[pl.ds(row0, ROWS_PER_TILE)], acc.at[pl.ds(row0, ROWS_PER_TILE)]
    )
    plsc.subcore_barrier()
    for k in range(2):
        wait_I(k, k)
        start_R(k, k)

    def group(g, _):
        for b0 in range(NBUF):
            j = g * NBUF + b0

            @pl.when(j >= 2)
            def _():
                wait_S(j - 2, (b0 - 2) % NBUF)

            @pl.when(j + 2 < N_CHUNKS)
            def _():
                wait_I(j + 2, (b0 + 2) % NBUF)

            @pl.when(j + 3 < N_CHUNKS)
            def _():
                start_I(j + 3, (b0 + 3) % NBUF)

            @pl.when(j + 2 < N_CHUNKS)
            def _():
                start_R(j + 2, (b0 + 2) % NBUF)

            wait_R(j, b0)
            relu_buf(b0)
            start_S(j, b0)
        return 0

    lax.fori_loop(0, N_CHUNKS // NBUF, group, 0)
    for j in (N_CHUNKS - 2, N_CHUNKS - 1):
        wait_S(j, j % NBUF)

    plsc.subcore_barrier()
    pltpu.sync_copy(
        acc.at[pl.ds(row0, ROWS_PER_TILE)],
        out_hbm.at[c, pl.ds(row0, ROWS_PER_TILE)],
    )


def _scatter_partials(E, dest, zeros):
    return pl.kernel(
        _scatter_body,
        out_type=jax.ShapeDtypeStruct((NC, N_PAD, H), jnp.float32),
        mesh=_mesh(),
        scratch_types=[
            pltpu.VMEM((NBUF, CHUNK), jnp.int32),
            pltpu.VMEM((NBUF, CHUNK, H), jnp.float32),
            pltpu.VMEM_SHARED((N_PAD, H), jnp.float32),
        ]
        + [pltpu.SemaphoreType.DMA] * (3 * NBUF),
    )(E, dest, zeros)


def _gather_body(a_hbm, np_hbm, src_hbm, rev_hbm, out_hbm, sidx, ridx, bufs, tbl,
                 *sems):
    semIs = sems[0:NBUF]
    semIr = sems[NBUF:2 * NBUF]
    semA = sems[2 * NBUF:3 * NBUF]
    semW = sems[3 * NBUF:4 * NBUF]
    semD = sems[4 * NBUF:]
    c = lax.axis_index("c")
    s = lax.axis_index("s")
    wid = c * NS + s
    row0 = s * ROWS_PER_TILE
    base = wid * EDGES_PER_TILE

    def out_dst(j):
        return out_hbm.at[pl.ds(base + j * CHUNK, CHUNK)]

    def start_Is(j, b):
        pltpu.async_copy(src_hbm.at[pl.ds(base + j * CHUNK, CHUNK)], sidx.at[b],
                         semIs[b])

    def wait_Is(j, b):
        pltpu.make_async_copy(src_hbm.at[pl.ds(base + j * CHUNK, CHUNK)],
                              sidx.at[b], semIs[b]).wait()

    def start_Ir(j, b):
        pltpu.async_copy(rev_hbm.at[pl.ds(base + j * CHUNK, CHUNK)], ridx.at[b],
                         semIr[b])

    def wait_Ir(j, b):
        pltpu.make_async_copy(rev_hbm.at[pl.ds(base + j * CHUNK, CHUNK)],
                              ridx.at[b], semIr[b]).wait()

    def start_A(j, b):
        pltpu.async_copy(tbl.at[sidx.at[b]], bufs.at[b], semA[b])

    def wait_A(j, b):
        pltpu.make_async_copy(tbl.at[sidx.at[b]], bufs.at[b], semA[b]).wait()

    def start_D(j, b):
        pltpu.async_copy(np_hbm.at[ridx.at[b]], bufs.at[b], semD[b], add=True)

    def wait_D(j, b):
        pltpu.make_async_copy(np_hbm.at[ridx.at[b]], bufs.at[b], semD[b]).wait()

    def start_W(j, b):
        pltpu.async_copy(bufs.at[b], out_dst(j), semW[b])

    def wait_W(j, b):
        pltpu.make_async_copy(bufs.at[b], out_dst(j), semW[b]).wait()

    for k in range(2):
        start_Is(k, k)
        start_Ir(k, k)
    pltpu.sync_copy(
        a_hbm.at[pl.ds(row0, ROWS_PER_TILE)], tbl.at[pl.ds(row0, ROWS_PER_TILE)]
    )
    plsc.subcore_barrier()
    wait_Is(0, 0)
    wait_Ir(0, 0)
    start_A(0, 0)

    def group(g, _):
        for b0 in range(NBUF):
            j = g * NBUF + b0

            @pl.when(j >= 4)
            def _():
                wait_W(j - 4, (b0 - 4) % NBUF)

            @pl.when(j >= 3)
            def _():
                wait_D(j - 3, (b0 - 3) % NBUF)
                start_W(j - 3, (b0 - 3) % NBUF)

            @pl.when(j + 1 < N_CHUNKS)
            def _():
                wait_Is(j + 1, (b0 + 1) % NBUF)
                wait_Ir(j + 1, (b0 + 1) % NBUF)

            @pl.when(j + 2 < N_CHUNKS)
            def _():
                start_Is(j + 2, (b0 + 2) % NBUF)
                start_Ir(j + 2, (b0 + 2) % NBUF)

            @pl.when(j + 1 < N_CHUNKS)
            def _():
                start_A(j + 1, (b0 + 1) % NBUF)

            wait_A(j, b0)
            start_D(j, b0)
        return 0

    lax.fori_loop(0, N_CHUNKS // NBUF, group, 0)
    for j in (N_CHUNKS - 3, N_CHUNKS - 2, N_CHUNKS - 1):
        wait_D(j, j % NBUF)
        start_W(j, j % NBUF)
    for j in range(N_CHUNKS - 4, N_CHUNKS):
        wait_W(j, j % NBUF)


def _gather_combine(A, nP, src, rev):
    return pl.kernel(
        _gather_body,
        out_type=jax.ShapeDtypeStruct((N_EDGES, H), jnp.float32),
        mesh=_mesh(),
        scratch_types=[
            pltpu.VMEM((NBUF, CHUNK), jnp.int32),
            pltpu.VMEM((NBUF, CHUNK), jnp.int32),
            pltpu.VMEM((NBUF, CHUNK, H), jnp.float32),
            pltpu.VMEM_SHARED((N_PAD, H), jnp.float32),
        ]
        + [pltpu.SemaphoreType.DMA] * (5 * NBUF),
    )(A, nP, src, rev)


def kernel(V, E, edge_index, rev_index, W, b):
    src = edge_index[0]
    dest = edge_index[1]
    zeros = jnp.zeros((N_PAD, H), jnp.float32)
    parts = _scatter_partials(E, dest, zeros)
    nP = _neg_relu_mm(E, W)
    A = _combine(parts, W, b)
    return _gather_combine(A, nP, src, rev_index)
